# baseline (speedup 1.0000x reference)
import functools

import jax
import jax.numpy as jnp
from jax import lax
from jax.experimental import pallas as pl
from jax.experimental.pallas import tpu as pltpu
from jax.experimental.pallas import tpu_sc as plsc

NUM_EMB = 1000000
D = 64
BATCH = 16384
HIST = 50
TOTAL = BATCH * HIST
NC = 2
NS = 16
NW = NC * NS
PER_W = TOTAL // NW
CHUNK = 128
NCHUNK = PER_W // CHUNK
NBT = BATCH // CHUNK
T = NCHUNK // 2

FULL_TC = 7812
MAXM = 245

_mesh = plsc.VectorSubcoreMesh(core_axis_name="c", subcore_axis_name="s")


@functools.partial(
    pl.kernel,
    mesh=_mesh,
    out_type=jax.ShapeDtypeStruct((NUM_EMB * D,), jnp.float32),
    scratch_types=[
        pltpu.VMEM((D, CHUNK), jnp.float32),
        pltpu.VMEM((D, CHUNK), jnp.float32),
        pltpu.VMEM((CHUNK * D,), jnp.float32),
        pltpu.VMEM((CHUNK * D,), jnp.float32),
        pltpu.SemaphoreType.DMA((2,)),
        pltpu.SemaphoreType.DMA((2,)),
    ],
    compiler_params=pltpu.CompilerParams(
        use_tc_tiling_on_sc=True, needs_layout_passes=False),
)
def _detile(wt_hbm, tail_hbm, lin_hbm, w_a, w_b, r_a, r_b, gsem, ssem):
    w_refs = (w_a, w_b)
    r_refs = (r_a, r_b)
    wid = lax.axis_index("s") * NC + lax.axis_index("c")
    start = 244 * wid + jnp.minimum(wid, 4)
    cnt = jnp.where(wid < 4, 245, 244)

    @pl.when(wid == NW - 1)
    def _():
        pltpu.sync_copy(tail_hbm, r_a.at[pl.ds(0, 4096)])
        pltpu.sync_copy(r_a.at[pl.ds(0, 4096)],
                        lin_hbm.at[pl.ds(8192 * FULL_TC, 4096)])

    iota = lax.iota(jnp.int32, 16)
    perms = [(iota + r) % 16 for r in range(16)]
    flats = [perms[r] * D + iota for r in range(16)]

    def fire_read(m, s):
        c = start + m
        pltpu.async_copy(
            wt_hbm.at[:, pl.ds(128 * c, 128)], w_refs[s], gsem.at[s])

    def wait_read(m, s):
        c = start + m
        pltpu.make_async_copy(
            wt_hbm.at[:, pl.ds(128 * c, 128)], w_refs[s], gsem.at[s]).wait()

    def transpose(s):
        def tb(gi, carry):
            l0 = gi * 16
            lvecs = [perms[r] + l0 for r in range(16)]
            fbase = [flats[r] + l0 * D for r in range(16)]
            for du in range(4):
                d0 = du * 16
                dvec = iota + d0
                for r in range(16):
                    vec = plsc.load_gather(w_refs[s], [dvec, lvecs[r]])
                    plsc.store_scatter(r_refs[s], [fbase[r] + d0], vec)
            return carry

        lax.fori_loop(0, 8, tb, 0)

    def fire_store(m, s):
        c = start + m
        pltpu.async_copy(
            r_refs[s], lin_hbm.at[pl.ds(8192 * c, 8192)], ssem.at[s])

    def drain_store(m, s):
        c = start + m
        pltpu.make_async_copy(
            r_refs[s], lin_hbm.at[pl.ds(8192 * c, 8192)], ssem.at[s]).wait()

    fire_read(0, 0)

    def body(t, carry):
        m0 = 2 * t
        m1 = 2 * t + 1

        @pl.when(m1 < cnt)
        def _():
            fire_read(m1, 1)

        @pl.when(m0 < cnt)
        def _():
            wait_read(m0, 0)

            @pl.when(t > 0)
            def _():
                drain_store(m0 - 2, 0)
            transpose(0)
            fire_store(m0, 0)

        @pl.when(m0 + 2 < cnt)
        def _():
            fire_read(m0 + 2, 0)

        @pl.when(m1 < cnt)
        def _():
            wait_read(m1, 1)

            @pl.when(t > 0)
            def _():
                drain_store(m1 - 2, 1)
            transpose(1)
            fire_store(m1, 1)

        return carry

    lax.fori_loop(0, (MAXM + 1) // 2, body, 0)

    def drain_last(m, s):

        @pl.when(s == 0)
        def _():
            drain_store(m, 0)

        @pl.when(s == 1)
        def _():
            drain_store(m, 1)

    drain_last(cnt - 2, (cnt - 2) % 2)
    drain_last(cnt - 1, (cnt - 1) % 2)


@functools.partial(
    pl.kernel,
    mesh=_mesh,
    out_type=jax.ShapeDtypeStruct((HIST, D // 8, NBT, 8, CHUNK), jnp.float32),
    scratch_types=[
        pltpu.VMEM((NCHUNK, CHUNK), jnp.int32),
        pltpu.VMEM((CHUNK, D), jnp.float32),
        pltpu.VMEM((CHUNK, D), jnp.float32),
        pltpu.VMEM((D, CHUNK), jnp.float32),
        pltpu.VMEM((D, CHUNK), jnp.float32),
        pltpu.SemaphoreType.DMA((2,)),
        pltpu.SemaphoreType.DMA((2,)),
    ],
    compiler_params=pltpu.CompilerParams(
        use_tc_tiling_on_sc=False, needs_layout_passes=False),
)
def _sc_gather(idx_hbm, table_hbm, out_hbm, idx_v, rows_a, rows_b, trans_a,
               trans_b, gsem, ssem):
    rows_refs = (rows_a, rows_b)
    trans_refs = (trans_a, trans_b)
    wid = lax.axis_index("s") * NC + lax.axis_index("c")
    cbase = wid * NCHUNK
    pltpu.sync_copy(idx_hbm.at[wid], idx_v)

    iota = lax.iota(jnp.int32, 16)
    perms = [(iota + r) % 16 for r in range(16)]

    def fire_gather(j, s):
        pltpu.async_copy(table_hbm.at[idx_v.at[j]], rows_refs[s], gsem.at[s])

    def wait_gather(j, s):
        pltpu.make_async_copy(
            table_hbm.at[idx_v.at[j]], rows_refs[s], gsem.at[s]).wait()

    def transpose(s):
        trans = trans_refs[s]

        def tb(gi, carry):
            b0 = gi * 16
            bvecs = [perms[r] + b0 for r in range(16)]
            for du in range(4):
                d0 = du * 16
                dvec = iota + d0
                for r in range(16):
                    vec = plsc.load_gather(rows_refs[s], [bvecs[r], dvec])
                    plsc.store_scatter(trans, [dvec, bvecs[r]], vec)
            return carry

        lax.fori_loop(0, 8, tb, 0)

    def fire_stores(j, s):
        c = cbase + j
        h = c // NBT
        bt = c - h * NBT
        for dt in range(8):
            pltpu.async_copy(
                trans_refs[s].at[pl.ds(dt * 8, 8), :],
                out_hbm.at[h, dt, bt], ssem.at[s])

    def drain_stores(s):
        for dt in range(8):
            pltpu.make_async_copy(
                trans_refs[s].at[pl.ds(dt * 8, 8), :],
                out_hbm.at[0, dt, 0], ssem.at[s]).wait()

    fire_gather(0, 0)

    def body(t, carry):
        j0 = 2 * t
        j1 = 2 * t + 1

        fire_gather(j1, 1)
        wait_gather(j0, 0)

        @pl.when(t > 0)
        def _():
            drain_stores(0)
        transpose(0)
        fire_stores(j0, 0)

        @pl.when(t < T - 1)
        def _():
            fire_gather(j0 + 2, 0)

        wait_gather(j1, 1)

        @pl.when(t > 0)
        def _():
            drain_stores(1)
        transpose(1)
        fire_stores(j1, 1)

        return carry

    lax.fori_loop(0, T, body, 0)
    drain_stores(0)
    drain_stores(1)


def kernel(input, weight):
    tail = weight[FULL_TC * 128:, :].reshape(4096)
    lin = _detile(weight.T, tail)
    table = lin.reshape(NUM_EMB, D)
    idx = input.astype(jnp.int32).T.reshape(NW, NCHUNK, CHUNK)
    out5 = _sc_gather(idx, table)
    return out5.transpose(2, 4, 0, 1, 3).reshape(BATCH, HIST, D)


# speedup vs baseline: 1.1387x; 1.1387x over previous
import functools

import jax
import jax.numpy as jnp
from jax import lax
from jax.experimental import pallas as pl
from jax.experimental.pallas import tpu as pltpu
from jax.experimental.pallas import tpu_sc as plsc

NUM_EMB = 1000000
D = 64
BATCH = 16384
HIST = 50
TOTAL = BATCH * HIST
NC = 2
NS = 16
NW = NC * NS
PER_W = TOTAL // NW
CHUNK = 128
NCHUNK = PER_W // CHUNK
NBT = BATCH // CHUNK
T = NCHUNK // 2

FULL_TC = 7812
MAXM = 245

_mesh = plsc.VectorSubcoreMesh(core_axis_name="c", subcore_axis_name="s")


@functools.partial(
    pl.kernel,
    mesh=_mesh,
    out_type=jax.ShapeDtypeStruct((NUM_EMB * D,), jnp.float32),
    scratch_types=[
        pltpu.VMEM((D, CHUNK), jnp.float32),
        pltpu.VMEM((D, CHUNK), jnp.float32),
        pltpu.VMEM((CHUNK * D,), jnp.float32),
        pltpu.VMEM((CHUNK * D,), jnp.float32),
        pltpu.SemaphoreType.DMA((2,)),
        pltpu.SemaphoreType.DMA((2,)),
    ],
    compiler_params=pltpu.CompilerParams(
        use_tc_tiling_on_sc=True, needs_layout_passes=False),
)
def _detile(wt_hbm, tail_hbm, lin_hbm, w_a, w_b, r_a, r_b, gsem, ssem):
    w_refs = (w_a, w_b)
    r_refs = (r_a, r_b)
    wid = lax.axis_index("s") * NC + lax.axis_index("c")
    start = 244 * wid + jnp.minimum(wid, 4)
    cnt = jnp.where(wid < 4, 245, 244)

    @pl.when(wid == NW - 1)
    def _():
        pltpu.sync_copy(tail_hbm, r_a.at[pl.ds(0, 4096)])
        pltpu.sync_copy(r_a.at[pl.ds(0, 4096)],
                        lin_hbm.at[pl.ds(8192 * FULL_TC, 4096)])

    iota = lax.iota(jnp.int32, 16)
    perms = [(iota + r) % 16 for r in range(16)]
    flats = [perms[r] * D + iota for r in range(16)]

    def fire_read(m, s):
        c = start + m
        pltpu.async_copy(
            wt_hbm.at[:, pl.ds(128 * c, 128)], w_refs[s], gsem.at[s])

    def wait_read(m, s):
        c = start + m
        pltpu.make_async_copy(
            wt_hbm.at[:, pl.ds(128 * c, 128)], w_refs[s], gsem.at[s]).wait()

    def transpose(s):
        def tb(gi, carry):
            l0 = gi * 16
            for du in range(4):
                d0 = du * 16
                dvec = iota + d0
                for r in range(16):
                    lvec = perms[r] + l0
                    vec = plsc.load_gather(w_refs[s], [dvec, lvec])
                    plsc.store_scatter(
                        r_refs[s], [flats[r] + (l0 * D + d0)], vec)
            return carry

        lax.fori_loop(0, 8, tb, 0)

    def fire_store(m, s):
        c = start + m
        pltpu.async_copy(
            r_refs[s], lin_hbm.at[pl.ds(8192 * c, 8192)], ssem.at[s])

    def drain_store(m, s):
        c = start + m
        pltpu.make_async_copy(
            r_refs[s], lin_hbm.at[pl.ds(8192 * c, 8192)], ssem.at[s]).wait()

    fire_read(0, 0)

    def body(t, carry):
        m0 = 2 * t
        m1 = 2 * t + 1

        @pl.when(m1 < cnt)
        def _():
            fire_read(m1, 1)

        @pl.when(m0 < cnt)
        def _():
            wait_read(m0, 0)

            @pl.when(t > 0)
            def _():
                drain_store(m0 - 2, 0)
            transpose(0)
            fire_store(m0, 0)

        @pl.when(m0 + 2 < cnt)
        def _():
            fire_read(m0 + 2, 0)

        @pl.when(m1 < cnt)
        def _():
            wait_read(m1, 1)

            @pl.when(t > 0)
            def _():
                drain_store(m1 - 2, 1)
            transpose(1)
            fire_store(m1, 1)

        return carry

    lax.fori_loop(0, (MAXM + 1) // 2, body, 0)

    def drain_last(m, s):

        @pl.when(s == 0)
        def _():
            drain_store(m, 0)

        @pl.when(s == 1)
        def _():
            drain_store(m, 1)

    drain_last(cnt - 2, (cnt - 2) % 2)
    drain_last(cnt - 1, (cnt - 1) % 2)


@functools.partial(
    pl.kernel,
    mesh=_mesh,
    out_type=jax.ShapeDtypeStruct((HIST, D // 8, NBT, 8, CHUNK), jnp.float32),
    scratch_types=[
        pltpu.VMEM((NCHUNK, CHUNK), jnp.int32),
        pltpu.VMEM((CHUNK, D), jnp.float32),
        pltpu.VMEM((CHUNK, D), jnp.float32),
        pltpu.VMEM((D, CHUNK), jnp.float32),
        pltpu.VMEM((D, CHUNK), jnp.float32),
        pltpu.SemaphoreType.DMA((2,)),
        pltpu.SemaphoreType.DMA((2,)),
    ],
    compiler_params=pltpu.CompilerParams(
        use_tc_tiling_on_sc=False, needs_layout_passes=False),
)
def _sc_gather(idx_hbm, table_hbm, out_hbm, idx_v, rows_a, rows_b, trans_a,
               trans_b, gsem, ssem):
    rows_refs = (rows_a, rows_b)
    trans_refs = (trans_a, trans_b)
    wid = lax.axis_index("s") * NC + lax.axis_index("c")
    cbase = wid * NCHUNK
    pltpu.sync_copy(idx_hbm.at[wid], idx_v)

    iota = lax.iota(jnp.int32, 16)
    perms = [(iota + r) % 16 for r in range(16)]

    def fire_gather(j, s):
        pltpu.async_copy(table_hbm.at[idx_v.at[j]], rows_refs[s], gsem.at[s])

    def wait_gather(j, s):
        pltpu.make_async_copy(
            table_hbm.at[idx_v.at[j]], rows_refs[s], gsem.at[s]).wait()

    def transpose(s):
        trans = trans_refs[s]

        def tb(gi, carry):
            b0 = gi * 16
            for du in range(4):
                d0 = du * 16
                dvec = iota + d0
                for r in range(16):
                    bvec = perms[r] + b0
                    vec = plsc.load_gather(rows_refs[s], [bvec, dvec])
                    plsc.store_scatter(trans, [dvec, bvec], vec)
            return carry

        lax.fori_loop(0, 8, tb, 0)

    def fire_stores(j, s):
        c = cbase + j
        h = c // NBT
        bt = c - h * NBT
        for dt in range(8):
            pltpu.async_copy(
                trans_refs[s].at[pl.ds(dt * 8, 8), :],
                out_hbm.at[h, dt, bt], ssem.at[s])

    def drain_stores(s):
        for dt in range(8):
            pltpu.make_async_copy(
                trans_refs[s].at[pl.ds(dt * 8, 8), :],
                out_hbm.at[0, dt, 0], ssem.at[s]).wait()

    fire_gather(0, 0)

    def body(t, carry):
        j0 = 2 * t
        j1 = 2 * t + 1

        fire_gather(j1, 1)
        wait_gather(j0, 0)

        @pl.when(t > 0)
        def _():
            drain_stores(0)
        transpose(0)
        fire_stores(j0, 0)

        @pl.when(t < T - 1)
        def _():
            fire_gather(j0 + 2, 0)

        wait_gather(j1, 1)

        @pl.when(t > 0)
        def _():
            drain_stores(1)
        transpose(1)
        fire_stores(j1, 1)

        return carry

    lax.fori_loop(0, T, body, 0)
    drain_stores(0)
    drain_stores(1)


def kernel(input, weight):
    tail = weight[FULL_TC * 128:, :].reshape(4096)
    lin = _detile(weight.T, tail)
    table = lin.reshape(NUM_EMB, D)
    idx = input.astype(jnp.int32).T.reshape(NW, NCHUNK, CHUNK)
    out5 = _sc_gather(idx, table)
    return out5.transpose(2, 4, 0, 1, 3).reshape(BATCH, HIST, D)


# speedup vs baseline: 1.1780x; 1.0345x over previous
import functools

import jax
import jax.numpy as jnp
from jax import lax
from jax.experimental import pallas as pl
from jax.experimental.pallas import tpu as pltpu
from jax.experimental.pallas import tpu_sc as plsc

NUM_EMB = 1000000
D = 64
BATCH = 16384
HIST = 50
TOTAL = BATCH * HIST
NC = 2
NS = 16
NW = NC * NS
PER_W = TOTAL // NW
CHUNK = 128
NCHUNK = PER_W // CHUNK
NBT = BATCH // CHUNK
T = NCHUNK // 2

FULL_TC = 7812
MAXM = 245

_mesh = plsc.VectorSubcoreMesh(core_axis_name="c", subcore_axis_name="s")


@functools.partial(
    pl.kernel,
    mesh=_mesh,
    out_type=jax.ShapeDtypeStruct((NUM_EMB * D,), jnp.float32),
    scratch_types=[
        pltpu.VMEM((D, CHUNK), jnp.float32),
        pltpu.VMEM((D, CHUNK), jnp.float32),
        pltpu.VMEM((CHUNK * D,), jnp.float32),
        pltpu.VMEM((CHUNK * D,), jnp.float32),
        pltpu.SemaphoreType.DMA((2,)),
        pltpu.SemaphoreType.DMA((2,)),
    ],
    compiler_params=pltpu.CompilerParams(
        use_tc_tiling_on_sc=True, needs_layout_passes=False),
)
def _detile(wt_hbm, tail_hbm, lin_hbm, w_a, w_b, r_a, r_b, gsem, ssem):
    w_refs = (w_a, w_b)
    r_refs = (r_a, r_b)
    wid = lax.axis_index("s") * NC + lax.axis_index("c")
    start = 244 * wid + jnp.minimum(wid, 4)
    cnt = jnp.where(wid < 4, 245, 244)

    @pl.when(wid == NW - 1)
    def _():
        pltpu.sync_copy(tail_hbm, r_a.at[pl.ds(0, 4096)])
        pltpu.sync_copy(r_a.at[pl.ds(0, 4096)],
                        lin_hbm.at[pl.ds(8192 * FULL_TC, 4096)])

    iota = lax.iota(jnp.int32, 16)
    perms = [(iota + r) % 16 for r in range(16)]
    flats = [perms[r] * D + iota for r in range(16)]

    def fire_read(m, s):
        c = start + m
        pltpu.async_copy(
            wt_hbm.at[:, pl.ds(128 * c, 128)], w_refs[s], gsem.at[s])

    def wait_read(m, s):
        c = start + m
        pltpu.make_async_copy(
            wt_hbm.at[:, pl.ds(128 * c, 128)], w_refs[s], gsem.at[s]).wait()

    def transpose(s):
        def tb(bi, carry):
            d0 = (bi % 4) * 16
            l0 = (bi // 4) * 16
            dvec = iota + d0
            for r in range(16):
                lvec = perms[r] + l0
                vec = plsc.load_gather(w_refs[s], [dvec, lvec])
                plsc.store_scatter(
                    r_refs[s], [flats[r] + (l0 * D + d0)], vec)
            return carry

        lax.fori_loop(0, 32, tb, 0)

    def fire_store(m, s):
        c = start + m
        pltpu.async_copy(
            r_refs[s], lin_hbm.at[pl.ds(8192 * c, 8192)], ssem.at[s])

    def drain_store(m, s):
        c = start + m
        pltpu.make_async_copy(
            r_refs[s], lin_hbm.at[pl.ds(8192 * c, 8192)], ssem.at[s]).wait()

    fire_read(0, 0)

    def body(t, carry):
        m0 = 2 * t
        m1 = 2 * t + 1

        @pl.when(m1 < cnt)
        def _():
            fire_read(m1, 1)

        @pl.when(m0 < cnt)
        def _():
            wait_read(m0, 0)

            @pl.when(t > 0)
            def _():
                drain_store(m0 - 2, 0)
            transpose(0)
            fire_store(m0, 0)

        @pl.when(m0 + 2 < cnt)
        def _():
            fire_read(m0 + 2, 0)

        @pl.when(m1 < cnt)
        def _():
            wait_read(m1, 1)

            @pl.when(t > 0)
            def _():
                drain_store(m1 - 2, 1)
            transpose(1)
            fire_store(m1, 1)

        return carry

    lax.fori_loop(0, (MAXM + 1) // 2, body, 0)

    def drain_last(m, s):

        @pl.when(s == 0)
        def _():
            drain_store(m, 0)

        @pl.when(s == 1)
        def _():
            drain_store(m, 1)

    drain_last(cnt - 2, (cnt - 2) % 2)
    drain_last(cnt - 1, (cnt - 1) % 2)


@functools.partial(
    pl.kernel,
    mesh=_mesh,
    out_type=jax.ShapeDtypeStruct((HIST, D // 8, NBT, 8, CHUNK), jnp.float32),
    scratch_types=[
        pltpu.VMEM((NCHUNK, CHUNK), jnp.int32),
        pltpu.VMEM((CHUNK, D), jnp.float32),
        pltpu.VMEM((CHUNK, D), jnp.float32),
        pltpu.VMEM((D, CHUNK), jnp.float32),
        pltpu.VMEM((D, CHUNK), jnp.float32),
        pltpu.SemaphoreType.DMA((2,)),
        pltpu.SemaphoreType.DMA((2,)),
    ],
    compiler_params=pltpu.CompilerParams(
        use_tc_tiling_on_sc=False, needs_layout_passes=False),
)
def _sc_gather(idx_hbm, table_hbm, out_hbm, idx_v, rows_a, rows_b, trans_a,
               trans_b, gsem, ssem):
    rows_refs = (rows_a, rows_b)
    trans_refs = (trans_a, trans_b)
    wid = lax.axis_index("s") * NC + lax.axis_index("c")
    cbase = wid * NCHUNK
    pltpu.sync_copy(idx_hbm.at[wid], idx_v)

    iota = lax.iota(jnp.int32, 16)
    perms = [(iota + r) % 16 for r in range(16)]

    def fire_gather(j, s):
        pltpu.async_copy(table_hbm.at[idx_v.at[j]], rows_refs[s], gsem.at[s])

    def wait_gather(j, s):
        pltpu.make_async_copy(
            table_hbm.at[idx_v.at[j]], rows_refs[s], gsem.at[s]).wait()

    def transpose(s):
        trans = trans_refs[s]

        def tb(gi, carry):
            b0 = gi * 16
            for du in range(4):
                d0 = du * 16
                dvec = iota + d0
                for r in range(16):
                    bvec = perms[r] + b0
                    vec = plsc.load_gather(rows_refs[s], [bvec, dvec])
                    plsc.store_scatter(trans, [dvec, bvec], vec)
            return carry

        lax.fori_loop(0, 8, tb, 0)

    def fire_stores(j, s):
        c = cbase + j
        h = c // NBT
        bt = c - h * NBT
        for dt in range(8):
            pltpu.async_copy(
                trans_refs[s].at[pl.ds(dt * 8, 8), :],
                out_hbm.at[h, dt, bt], ssem.at[s])

    def drain_stores(s):
        for dt in range(8):
            pltpu.make_async_copy(
                trans_refs[s].at[pl.ds(dt * 8, 8), :],
                out_hbm.at[0, dt, 0], ssem.at[s]).wait()

    fire_gather(0, 0)

    def body(t, carry):
        j0 = 2 * t
        j1 = 2 * t + 1

        fire_gather(j1, 1)
        wait_gather(j0, 0)

        @pl.when(t > 0)
        def _():
            drain_stores(0)
        transpose(0)
        fire_stores(j0, 0)

        @pl.when(t < T - 1)
        def _():
            fire_gather(j0 + 2, 0)

        wait_gather(j1, 1)

        @pl.when(t > 0)
        def _():
            drain_stores(1)
        transpose(1)
        fire_stores(j1, 1)

        return carry

    lax.fori_loop(0, T, body, 0)
    drain_stores(0)
    drain_stores(1)


def kernel(input, weight):
    tail = weight[FULL_TC * 128:, :].reshape(4096)
    lin = _detile(weight.T, tail)
    table = lin.reshape(NUM_EMB, D)
    idx = input.astype(jnp.int32).T.reshape(NW, NCHUNK, CHUNK)
    out5 = _sc_gather(idx, table)
    return out5.transpose(2, 4, 0, 1, 3).reshape(BATCH, HIST, D)
